# baseline (speedup 1.0000x reference)
import functools

import jax
import jax.numpy as jnp
from jax import lax
from jax.experimental import pallas as pl
from jax.experimental.pallas import tpu as pltpu
from jax.experimental.pallas import tpu_sc as plsc

_VOCAB_TILE = 1024
_L = 16


def _sc_info():
    try:
        info = plsc.get_sparse_core_info()
        return info.num_cores, info.num_subcores
    except Exception:
        return 2, 16


@functools.cache
def _make_gather_add(n, T, D):
    NC, NS = _sc_info()
    NW = NC * NS
    per_w = n // NW
    assert n % NW == 0 and per_w % _L == 0 and per_w % T == 0
    assert D == 2 * _L
    lines_w = per_w * D // 128
    mesh = plsc.VectorSubcoreMesh(core_axis_name="c", subcore_axis_name="s")

    @functools.partial(
        pl.kernel,
        mesh=mesh,
        out_type=jax.ShapeDtypeStruct((n * D // 128, 128), jnp.float32),
        scratch_types=[
            pltpu.VMEM((per_w,), jnp.int32),
            pltpu.VMEM((per_w,), jnp.int32),
            pltpu.VMEM((per_w,), jnp.int32),
            pltpu.VMEM((per_w, _L), jnp.float32),
            pltpu.VMEM((per_w, _L), jnp.float32),
            pltpu.VMEM((T, D), jnp.float32),
            pltpu.VMEM((lines_w, 128), jnp.float32),
            pltpu.SemaphoreType.DMA,
        ],
        compiler_params=pltpu.CompilerParams(use_tc_tiling_on_sc=False),
    )
    def gather_add(idx_hbm, tok16_hbm, pos_hbm, x_hbm,
                   idx_v, ev_v, od_v, g_ev, g_od, pos_v, x_s, sem):
        wid = lax.axis_index("s") * NC + lax.axis_index("c")
        base = wid * per_w
        pltpu.sync_copy(idx_hbm.at[pl.ds(base, per_w)], idx_v)
        pltpu.sync_copy(pos_hbm, pos_v)
        for a in range(per_w // _L):
            i16 = idx_v[pl.ds(a * _L, _L)]
            ev_v[pl.ds(a * _L, _L)] = i16 * 2
            od_v[pl.ds(a * _L, _L)] = i16 * 2 + 1
        cp_ev = pltpu.async_copy(tok16_hbm.at[ev_v], g_ev, sem)
        cp_od = pltpu.async_copy(tok16_hbm.at[od_v], g_od, sem)
        cp_ev.wait()
        cp_od.wait()
        for i in range(per_w):
            r, l = (i * D) // 128, (i * D) % 128
            x_s[r, pl.ds(l, _L)] = g_ev[i, :] + pos_v[i % T, pl.ds(0, _L)]
            x_s[r, pl.ds(l + _L, _L)] = g_od[i, :] + pos_v[i % T, pl.ds(_L, _L)]
        pltpu.sync_copy(x_s, x_hbm.at[pl.ds(wid * lines_w, lines_w)])

    return gather_add


def _mm_body(x_ref, w_ref, b_ref, o_ref):
    o_ref[...] = (
        jnp.dot(x_ref[...], w_ref[...], preferred_element_type=jnp.float32)
        + b_ref[...]
    )


def _matmul_bias(x, W, b2d):
    n, d = x.shape
    V = W.shape[1]
    vt = _VOCAB_TILE
    return pl.pallas_call(
        _mm_body,
        grid=(pl.cdiv(V, vt),),
        in_specs=[
            pl.BlockSpec((n, d), lambda j: (0, 0)),
            pl.BlockSpec((d, vt), lambda j: (0, j)),
            pl.BlockSpec((1, vt), lambda j: (0, j)),
        ],
        out_specs=pl.BlockSpec((n, vt), lambda j: (0, j)),
        out_shape=jax.ShapeDtypeStruct((n, V), jnp.float32),
        compiler_params=pltpu.CompilerParams(
            dimension_semantics=("arbitrary",)
        ),
    )(x, W, b2d)


def kernel(idx, tok_table, pos_table, W, b):
    B, T = idx.shape
    V, D = tok_table.shape
    n = B * T
    tok16 = tok_table.reshape(V * D // _L, _L)
    x_packed = _make_gather_add(n, T, D)(
        idx.reshape(n), tok16, pos_table[:T].astype(jnp.float32)
    )
    x = x_packed.reshape(n, D)
    logits = _matmul_bias(x, W, b.reshape(1, V))
    return logits.reshape(B, T, V)


# speedup vs baseline: 1.1130x; 1.1130x over previous
import functools

import jax
import jax.numpy as jnp
from jax import lax
from jax.experimental import pallas as pl
from jax.experimental.pallas import tpu as pltpu
from jax.experimental.pallas import tpu_sc as plsc

_VOCAB_TILE = 1024
_L = 16
_CHUNK = 128


def _sc_info():
    try:
        info = plsc.get_sparse_core_info()
        return info.num_cores, info.num_subcores
    except Exception:
        return 2, 16


@functools.cache
def _make_gather_t(n, T, D, V):
    NC, NS = _sc_info()
    NW = NC * NS
    B = n // T
    slab_c = (D * T) // NW
    per_w = B * slab_c
    assert per_w % _CHUNK == 0 and B % _L == 0 and NW % T == 0
    w_per_t = NW // T
    mesh = plsc.VectorSubcoreMesh(core_axis_name="c", subcore_axis_name="s")

    @functools.partial(
        pl.kernel,
        mesh=mesh,
        out_type=jax.ShapeDtypeStruct((n * D,), jnp.float32),
        scratch_types=[
            pltpu.VMEM((B,), jnp.int32),
            pltpu.VMEM((per_w,), jnp.int32),
            pltpu.VMEM((per_w,), jnp.float32),
            pltpu.SemaphoreType.DMA,
        ],
        compiler_params=pltpu.CompilerParams(use_tc_tiling_on_sc=False),
    )
    def gather_t(idx_hbm, tok1_hbm, x_hbm, idx_v, ivec, x_s, sem):
        wid = lax.axis_index("s") * NC + lax.axis_index("c")
        t = wid // w_per_t
        c0 = (wid % w_per_t) * slab_c
        pltpu.sync_copy(idx_hbm.at[pl.ds(t * B, B)], idx_v)
        for cc in range(slab_c):
            cbase = (c0 + cc) * V
            for a in range(B // _L):
                i16 = idx_v[pl.ds(a * _L, _L)]
                ivec[pl.ds(cc * B + a * _L, _L)] = i16 + cbase
        copies = [
            pltpu.async_copy(
                tok1_hbm.at[ivec.at[pl.ds(k * _CHUNK, _CHUNK)]],
                x_s.at[pl.ds(k * _CHUNK, _CHUNK)],
                sem,
            )
            for k in range(per_w // _CHUNK)
        ]
        for cp in copies:
            cp.wait()
        pltpu.sync_copy(x_s, x_hbm.at[pl.ds(wid * per_w, per_w)])

    return gather_t


def _mm_body(wt_ref, x_ref, pos_ref, b_ref, o_ref):
    acc = lax.dot_general(
        wt_ref[...], x_ref[0],
        dimension_numbers=(((1,), (0,)), ((), ())),
        preferred_element_type=jnp.float32,
    )
    pos_row = pos_ref[pl.ds(pl.program_id(1), 1), :]
    pw = lax.dot_general(
        wt_ref[...], pos_row,
        dimension_numbers=(((1,), (1,)), ((), ())),
        preferred_element_type=jnp.float32,
    )
    o_ref[0] = acc + pw + b_ref[...]


def _matmul_t(Wt, xT, pos, b2d):
    V, d = Wt.shape
    T, _, B = xT.shape
    vt = _VOCAB_TILE
    return pl.pallas_call(
        _mm_body,
        grid=(pl.cdiv(V, vt), T),
        in_specs=[
            pl.BlockSpec((vt, d), lambda j, t: (j, 0)),
            pl.BlockSpec((1, d, B), lambda j, t: (t, 0, 0)),
            pl.BlockSpec((T, d), lambda j, t: (0, 0)),
            pl.BlockSpec((vt, 1), lambda j, t: (j, 0)),
        ],
        out_specs=pl.BlockSpec((1, vt, B), lambda j, t: (t, j, 0)),
        out_shape=jax.ShapeDtypeStruct((T, V, B), jnp.float32),
        compiler_params=pltpu.CompilerParams(
            dimension_semantics=("arbitrary", "arbitrary"),
            fuse_transposed_lhs_in_matmul=True,
        ),
    )(Wt, xT, pos, b2d)


def kernel(idx, tok_table, pos_table, W, b):
    B, T = idx.shape
    V, D = tok_table.shape
    n = B * T
    idx_t = idx.T.reshape(n)
    tok1 = tok_table.T.reshape(V * D)
    x_flat = _make_gather_t(n, T, D, V)(idx_t, tok1)
    xT = x_flat.reshape(T, D, B)
    out = _matmul_t(
        W.T, xT, pos_table[:T].astype(jnp.float32), b.reshape(V, 1)
    )
    return jnp.transpose(out, (2, 0, 1))


# speedup vs baseline: 1.1135x; 1.0004x over previous
import functools

import jax
import jax.numpy as jnp
from jax import lax
from jax.experimental import pallas as pl
from jax.experimental.pallas import tpu as pltpu
from jax.experimental.pallas import tpu_sc as plsc

_VOCAB_TILE = 1024
_L = 16
_CHUNK = 128


def _sc_info():
    try:
        info = plsc.get_sparse_core_info()
        return info.num_cores, info.num_subcores
    except Exception:
        return 2, 16


@functools.cache
def _make_gather_t(n, T, D, V):
    NC, NS = _sc_info()
    NW = NC * NS
    B = n // T
    slab_c = (D * T) // NW
    per_w = B * slab_c
    assert per_w % _CHUNK == 0 and B % _L == 0 and NW % T == 0
    w_per_t = NW // T
    mesh = plsc.VectorSubcoreMesh(core_axis_name="c", subcore_axis_name="s")

    @functools.partial(
        pl.kernel,
        mesh=mesh,
        out_type=jax.ShapeDtypeStruct((n * D,), jnp.float32),
        scratch_types=[
            pltpu.VMEM((B,), jnp.int32),
            pltpu.VMEM((per_w,), jnp.int32),
            pltpu.VMEM((per_w,), jnp.float32),
            pltpu.SemaphoreType.DMA,
        ],
        compiler_params=pltpu.CompilerParams(use_tc_tiling_on_sc=False),
    )
    def gather_t(idx_hbm, tok1_hbm, x_hbm, idx_v, ivec, x_s, sem):
        wid = lax.axis_index("s") * NC + lax.axis_index("c")
        t = wid // w_per_t
        c0 = (wid % w_per_t) * slab_c
        pltpu.sync_copy(idx_hbm.at[pl.ds(t * B, B)], idx_v)
        for cc in range(slab_c):
            cbase = (c0 + cc) * V
            for a in range(B // _L):
                i16 = idx_v[pl.ds(a * _L, _L)]
                ivec[pl.ds(cc * B + a * _L, _L)] = i16 + cbase
        copies = [
            pltpu.async_copy(
                tok1_hbm.at[ivec.at[pl.ds(k * _CHUNK, _CHUNK)]],
                x_s.at[pl.ds(k * _CHUNK, _CHUNK)],
                sem,
            )
            for k in range(per_w // _CHUNK)
        ]
        for cp in copies:
            cp.wait()
        pltpu.sync_copy(x_s, x_hbm.at[pl.ds(wid * per_w, per_w)])

    return gather_t


def _mm_body(wt_ref, x_ref, pos_ref, b_ref, o_ref):
    acc = lax.dot_general(
        wt_ref[...], x_ref[0],
        dimension_numbers=(((1,), (0,)), ((), ())),
        preferred_element_type=jnp.float32,
    )
    pos_row = pos_ref[pl.ds(pl.program_id(1), 1), :]
    pw = lax.dot_general(
        wt_ref[...], pos_row,
        dimension_numbers=(((1,), (1,)), ((), ())),
        preferred_element_type=jnp.float32,
    )
    o_ref[0] = acc + pw + b_ref[...]


def _matmul_t(Wt, xT, pos, b2d):
    V, d = Wt.shape
    T, _, B = xT.shape
    vt = _VOCAB_TILE
    return pl.pallas_call(
        _mm_body,
        grid=(pl.cdiv(V, vt), T),
        in_specs=[
            pl.BlockSpec((vt, d), lambda j, t: (j, 0)),
            pl.BlockSpec((1, d, B), lambda j, t: (t, 0, 0)),
            pl.BlockSpec((T, d), lambda j, t: (0, 0)),
            pl.BlockSpec((vt, 1), lambda j, t: (j, 0)),
        ],
        out_specs=pl.BlockSpec((1, vt, B), lambda j, t: (t, j, 0)),
        out_shape=jax.ShapeDtypeStruct((T, V, B), jnp.float32),
        compiler_params=pltpu.CompilerParams(
            dimension_semantics=("parallel", "parallel"),
            fuse_transposed_lhs_in_matmul=True,
        ),
    )(Wt, xT, pos, b2d)


def kernel(idx, tok_table, pos_table, W, b):
    B, T = idx.shape
    V, D = tok_table.shape
    n = B * T
    idx_t = idx.T.reshape(n)
    tok1 = tok_table.T.reshape(V * D)
    x_flat = _make_gather_t(n, T, D, V)(idx_t, tok1)
    xT = x_flat.reshape(T, D, B)
    out = _matmul_t(
        W.T, xT, pos_table[:T].astype(jnp.float32), b.reshape(V, 1)
    )
    return jnp.transpose(out, (2, 0, 1))


# speedup vs baseline: 1.1805x; 1.0602x over previous
import functools

import jax
import jax.numpy as jnp
from jax import lax
from jax.experimental import pallas as pl
from jax.experimental.pallas import tpu as pltpu
from jax.experimental.pallas import tpu_sc as plsc

_VOCAB_TILE = 1000
_L = 16
_CHUNK = 128


def _sc_info():
    try:
        info = plsc.get_sparse_core_info()
        return info.num_cores, info.num_subcores
    except Exception:
        return 2, 16


@functools.cache
def _make_gather_t(n, T, D, V):
    NC, NS = _sc_info()
    NW = NC * NS
    B = n // T
    slab_c = (D * T) // NW
    per_w = B * slab_c
    assert per_w % _CHUNK == 0 and B % _L == 0 and NW % T == 0
    w_per_t = NW // T
    mesh = plsc.VectorSubcoreMesh(core_axis_name="c", subcore_axis_name="s")

    @functools.partial(
        pl.kernel,
        mesh=mesh,
        out_type=jax.ShapeDtypeStruct((n * D,), jnp.float32),
        scratch_types=[
            pltpu.VMEM((B,), jnp.int32),
            pltpu.VMEM((per_w,), jnp.int32),
            pltpu.VMEM((per_w,), jnp.float32),
            pltpu.SemaphoreType.DMA,
        ],
        compiler_params=pltpu.CompilerParams(use_tc_tiling_on_sc=False),
    )
    def gather_t(idx_hbm, tok1_hbm, x_hbm, idx_v, ivec, x_s, sem):
        wid = lax.axis_index("s") * NC + lax.axis_index("c")
        t = wid // w_per_t
        c0 = (wid % w_per_t) * slab_c
        pltpu.sync_copy(idx_hbm.at[pl.ds(t * B, B)], idx_v)
        for cc in range(slab_c):
            cbase = (c0 + cc) * V
            for a in range(B // _L):
                i16 = idx_v[pl.ds(a * _L, _L)]
                ivec[pl.ds(cc * B + a * _L, _L)] = i16 + cbase
        copies = [
            pltpu.async_copy(
                tok1_hbm.at[ivec.at[pl.ds(k * _CHUNK, _CHUNK)]],
                x_s.at[pl.ds(k * _CHUNK, _CHUNK)],
                sem,
            )
            for k in range(per_w // _CHUNK)
        ]
        for cp in copies:
            cp.wait()
        pltpu.sync_copy(x_s, x_hbm.at[pl.ds(wid * per_w, per_w)])

    return gather_t


def _mm_body(wt_ref, x_ref, pos_ref, b_ref, o_ref):
    t = pl.program_id(1)
    wt = wt_ref[...]
    acc = lax.dot_general(
        wt, x_ref[t],
        dimension_numbers=(((1,), (0,)), ((), ())),
        preferred_element_type=jnp.float32,
    )
    pos_row = pos_ref[pl.ds(t, 1), :]
    pw = lax.dot_general(
        wt, pos_row,
        dimension_numbers=(((1,), (1,)), ((), ())),
        preferred_element_type=jnp.float32,
    )
    o_ref[0] = acc + pw + b_ref[...]


def _matmul_t(Wt, xT, pos, b2d):
    V, d = Wt.shape
    T, _, B = xT.shape
    vt = _VOCAB_TILE
    return pl.pallas_call(
        _mm_body,
        grid=(pl.cdiv(V, vt), T),
        in_specs=[
            pl.BlockSpec((vt, d), lambda j, t: (j, 0)),
            pl.BlockSpec((T, d, B), lambda j, t: (0, 0, 0)),
            pl.BlockSpec((T, d), lambda j, t: (0, 0)),
            pl.BlockSpec((vt, 1), lambda j, t: (j, 0)),
        ],
        out_specs=pl.BlockSpec((1, vt, B), lambda j, t: (t, j, 0)),
        out_shape=jax.ShapeDtypeStruct((T, V, B), jnp.float32),
        compiler_params=pltpu.CompilerParams(
            dimension_semantics=("parallel", "parallel"),
            fuse_transposed_lhs_in_matmul=True,
        ),
    )(Wt, xT, pos, b2d)


def kernel(idx, tok_table, pos_table, W, b):
    B, T = idx.shape
    V, D = tok_table.shape
    n = B * T
    idx_t = idx.T.reshape(n)
    tok1 = tok_table.T.reshape(V * D)
    x_flat = _make_gather_t(n, T, D, V)(idx_t, tok1)
    xT = x_flat.reshape(T, D, B)
    out = _matmul_t(
        W.T, xT, pos_table[:T].astype(jnp.float32), b.reshape(V, 1)
    )
    return jnp.transpose(out, (2, 0, 1))


# speedup vs baseline: 2.4328x; 2.0608x over previous
import functools

import jax
import jax.numpy as jnp
from jax import lax
from jax.experimental import pallas as pl
from jax.experimental.pallas import tpu as pltpu
from jax.experimental.pallas import tpu_sc as plsc

_VOCAB_TILE = 1000
_L = 16
_CHUNK = 128


def _sc_info():
    try:
        info = plsc.get_sparse_core_info()
        return info.num_cores, info.num_subcores
    except Exception:
        return 2, 16


@functools.cache
def _make_gather_t(n, T, D, V):
    NC, NS = _sc_info()
    NW = NC * NS
    B = n // T
    slab_c = (D * T) // NW
    per_w = B * slab_c
    assert per_w % _CHUNK == 0 and B % _L == 0 and NW % T == 0
    w_per_t = NW // T
    mesh = plsc.VectorSubcoreMesh(core_axis_name="c", subcore_axis_name="s")

    @functools.partial(
        pl.kernel,
        mesh=mesh,
        out_type=jax.ShapeDtypeStruct((n * D,), jnp.float32),
        scratch_types=[
            pltpu.VMEM((B,), jnp.int32),
            pltpu.VMEM((per_w,), jnp.int32),
            pltpu.VMEM((per_w,), jnp.float32),
            pltpu.SemaphoreType.DMA,
        ],
        compiler_params=pltpu.CompilerParams(use_tc_tiling_on_sc=False),
    )
    def gather_t(idx_hbm, tok1_hbm, x_hbm, idx_v, ivec, x_s, sem):
        wid = lax.axis_index("s") * NC + lax.axis_index("c")
        t = wid // w_per_t
        c0 = (wid % w_per_t) * slab_c
        pltpu.sync_copy(idx_hbm.at[pl.ds(t * B, B)], idx_v)
        for cc in range(slab_c):
            cbase = (c0 + cc) * V
            for a in range(B // _L):
                i16 = idx_v[pl.ds(a * _L, _L)]
                ivec[pl.ds(cc * B + a * _L, _L)] = i16 + cbase
        copies = [
            pltpu.async_copy(
                tok1_hbm.at[ivec.at[pl.ds(k * _CHUNK, _CHUNK)]],
                x_s.at[pl.ds(k * _CHUNK, _CHUNK)],
                sem,
            )
            for k in range(per_w // _CHUNK)
        ]
        for cp in copies:
            cp.wait()
        pltpu.sync_copy(x_s, x_hbm.at[pl.ds(wid * per_w, per_w)])

    return gather_t


def _mm_body(wt_ref, x_ref, pos_ref, b_ref, o_ref):
    wt = wt_ref[...]
    bias = b_ref[...]
    for t in range(o_ref.shape[0]):
        acc = lax.dot_general(
            wt, x_ref[t],
            dimension_numbers=(((1,), (0,)), ((), ())),
            preferred_element_type=jnp.float32,
        )
        pw = lax.dot_general(
            wt, pos_ref[pl.ds(t, 1), :],
            dimension_numbers=(((1,), (1,)), ((), ())),
            preferred_element_type=jnp.float32,
        )
        o_ref[t] = acc + pw + bias


def _matmul_t(Wt, xT, pos, b2d):
    V, d = Wt.shape
    T, _, B = xT.shape
    vt = _VOCAB_TILE
    return pl.pallas_call(
        _mm_body,
        grid=(pl.cdiv(V, vt),),
        in_specs=[
            pl.BlockSpec((vt, d), lambda j: (j, 0)),
            pl.BlockSpec((T, d, B), lambda j: (0, 0, 0)),
            pl.BlockSpec((T, d), lambda j: (0, 0)),
            pl.BlockSpec((vt, 1), lambda j: (j, 0)),
        ],
        out_specs=pl.BlockSpec((T, vt, B), lambda j: (0, j, 0)),
        out_shape=jax.ShapeDtypeStruct((T, V, B), jnp.float32),
        compiler_params=pltpu.CompilerParams(
            dimension_semantics=("parallel",),
            fuse_transposed_lhs_in_matmul=True,
        ),
    )(Wt, xT, pos, b2d)


def kernel(idx, tok_table, pos_table, W, b):
    B, T = idx.shape
    V, D = tok_table.shape
    n = B * T
    idx_t = idx.T.reshape(n)
    tok1 = tok_table.T.reshape(V * D)
    x_flat = _make_gather_t(n, T, D, V)(idx_t, tok1)
    xT = x_flat.reshape(T, D, B)
    out = _matmul_t(
        W.T, xT, pos_table[:T].astype(jnp.float32), b.reshape(V, 1)
    )
    return jnp.transpose(out, (2, 0, 1))


# speedup vs baseline: 2.4763x; 1.0179x over previous
import functools

import jax
import jax.numpy as jnp
from jax import lax
from jax.experimental import pallas as pl
from jax.experimental.pallas import tpu as pltpu
from jax.experimental.pallas import tpu_sc as plsc

_VOCAB_TILE = 2000
_L = 16
_CHUNK = 128


def _sc_info():
    try:
        info = plsc.get_sparse_core_info()
        return info.num_cores, info.num_subcores
    except Exception:
        return 2, 16


@functools.cache
def _make_gather_t(n, T, D, V):
    NC, NS = _sc_info()
    NW = NC * NS
    B = n // T
    slab_c = (D * T) // NW
    per_w = B * slab_c
    assert per_w % _CHUNK == 0 and B % _L == 0 and NW % T == 0
    w_per_t = NW // T
    mesh = plsc.VectorSubcoreMesh(core_axis_name="c", subcore_axis_name="s")

    @functools.partial(
        pl.kernel,
        mesh=mesh,
        out_type=jax.ShapeDtypeStruct((n * D,), jnp.float32),
        scratch_types=[
            pltpu.VMEM((B,), jnp.int32),
            pltpu.VMEM((per_w,), jnp.int32),
            pltpu.VMEM((per_w,), jnp.float32),
            pltpu.SemaphoreType.DMA,
        ],
        compiler_params=pltpu.CompilerParams(use_tc_tiling_on_sc=False),
    )
    def gather_t(idx_hbm, tok1_hbm, x_hbm, idx_v, ivec, x_s, sem):
        wid = lax.axis_index("s") * NC + lax.axis_index("c")
        t = wid // w_per_t
        c0 = (wid % w_per_t) * slab_c
        pltpu.sync_copy(idx_hbm.at[pl.ds(t * B, B)], idx_v)
        for cc in range(slab_c):
            cbase = (c0 + cc) * V
            for a in range(B // _L):
                i16 = idx_v[pl.ds(a * _L, _L)]
                ivec[pl.ds(cc * B + a * _L, _L)] = i16 + cbase
        copies = [
            pltpu.async_copy(
                tok1_hbm.at[ivec.at[pl.ds(k * _CHUNK, _CHUNK)]],
                x_s.at[pl.ds(k * _CHUNK, _CHUNK)],
                sem,
            )
            for k in range(per_w // _CHUNK)
        ]
        for cp in copies:
            cp.wait()
        pltpu.sync_copy(x_s, x_hbm.at[pl.ds(wid * per_w, per_w)])

    return gather_t


def _mm_body(wt_ref, x_ref, pos_ref, b_ref, o_ref):
    wt = wt_ref[...]
    bias = b_ref[...]
    for t in range(o_ref.shape[0]):
        acc = lax.dot_general(
            wt, x_ref[t],
            dimension_numbers=(((1,), (0,)), ((), ())),
            preferred_element_type=jnp.float32,
        )
        pw = lax.dot_general(
            wt, pos_ref[pl.ds(t, 1), :],
            dimension_numbers=(((1,), (1,)), ((), ())),
            preferred_element_type=jnp.float32,
        )
        o_ref[t] = acc + pw + bias


def _matmul_t(Wt, xT, pos, b2d):
    V, d = Wt.shape
    T, _, B = xT.shape
    vt = _VOCAB_TILE
    return pl.pallas_call(
        _mm_body,
        grid=(pl.cdiv(V, vt),),
        in_specs=[
            pl.BlockSpec((vt, d), lambda j: (j, 0)),
            pl.BlockSpec((T, d, B), lambda j: (0, 0, 0)),
            pl.BlockSpec((T, d), lambda j: (0, 0)),
            pl.BlockSpec((vt, 1), lambda j: (j, 0)),
        ],
        out_specs=pl.BlockSpec((T, vt, B), lambda j: (0, j, 0)),
        out_shape=jax.ShapeDtypeStruct((T, V, B), jnp.float32),
        compiler_params=pltpu.CompilerParams(
            dimension_semantics=("parallel",),
            fuse_transposed_lhs_in_matmul=True,
        ),
    )(Wt, xT, pos, b2d)


def kernel(idx, tok_table, pos_table, W, b):
    B, T = idx.shape
    V, D = tok_table.shape
    n = B * T
    idx_t = idx.T.reshape(n)
    tok1 = tok_table.T.reshape(V * D)
    x_flat = _make_gather_t(n, T, D, V)(idx_t, tok1)
    xT = x_flat.reshape(T, D, B)
    out = _matmul_t(
        W.T, xT, pos_table[:T].astype(jnp.float32), b.reshape(V, 1)
    )
    return jnp.transpose(out, (2, 0, 1))


# speedup vs baseline: 2.5806x; 1.0421x over previous
import functools

import jax
import jax.numpy as jnp
from jax import lax
from jax.experimental import pallas as pl
from jax.experimental.pallas import tpu as pltpu
from jax.experimental.pallas import tpu_sc as plsc

_VOCAB_TILE = 2048
_L = 16
_CHUNK = 128


def _sc_info():
    try:
        info = plsc.get_sparse_core_info()
        return info.num_cores, info.num_subcores
    except Exception:
        return 2, 16


@functools.cache
def _make_gather_t(n, T, D, V):
    NC, NS = _sc_info()
    NW = NC * NS
    B = n // T
    slab_c = (D * T) // NW
    per_w = B * slab_c
    assert per_w % _CHUNK == 0 and B % _L == 0 and NW % T == 0
    w_per_t = NW // T
    mesh = plsc.VectorSubcoreMesh(core_axis_name="c", subcore_axis_name="s")

    @functools.partial(
        pl.kernel,
        mesh=mesh,
        out_type=jax.ShapeDtypeStruct((n * D,), jnp.float32),
        scratch_types=[
            pltpu.VMEM((B,), jnp.int32),
            pltpu.VMEM((per_w,), jnp.int32),
            pltpu.VMEM((per_w,), jnp.float32),
            pltpu.SemaphoreType.DMA,
        ],
        compiler_params=pltpu.CompilerParams(use_tc_tiling_on_sc=False),
    )
    def gather_t(idx_hbm, tok1_hbm, x_hbm, idx_v, ivec, x_s, sem):
        wid = lax.axis_index("s") * NC + lax.axis_index("c")
        t = wid // w_per_t
        c0 = (wid % w_per_t) * slab_c
        pltpu.sync_copy(idx_hbm.at[pl.ds(t * B, B)], idx_v)
        for cc in range(slab_c):
            cbase = (c0 + cc) * V
            for a in range(B // _L):
                i16 = idx_v[pl.ds(a * _L, _L)]
                ivec[pl.ds(cc * B + a * _L, _L)] = i16 + cbase
        copies = [
            pltpu.async_copy(
                tok1_hbm.at[ivec.at[pl.ds(k * _CHUNK, _CHUNK)]],
                x_s.at[pl.ds(k * _CHUNK, _CHUNK)],
                sem,
            )
            for k in range(per_w // _CHUNK)
        ]
        for cp in copies:
            cp.wait()
        pltpu.sync_copy(x_s, x_hbm.at[pl.ds(wid * per_w, per_w)])

    return gather_t


def _mm_body(w_ref, x_ref, pos_ref, b_ref, o_ref):
    wt = lax.transpose(w_ref[...], (1, 0))
    bias = b_ref[...]
    for t in range(o_ref.shape[0]):
        acc = lax.dot_general(
            wt, x_ref[t],
            dimension_numbers=(((1,), (0,)), ((), ())),
            preferred_element_type=jnp.float32,
        )
        pw = lax.dot_general(
            wt, pos_ref[pl.ds(t, 1), :],
            dimension_numbers=(((1,), (1,)), ((), ())),
            preferred_element_type=jnp.float32,
        )
        o_ref[t] = acc + pw + bias


def _matmul_t(W, xT, pos, b2d):
    d, V = W.shape
    T, _, B = xT.shape
    vt = _VOCAB_TILE
    return pl.pallas_call(
        _mm_body,
        grid=(pl.cdiv(V, vt),),
        in_specs=[
            pl.BlockSpec((d, vt), lambda j: (0, j)),
            pl.BlockSpec((T, d, B), lambda j: (0, 0, 0)),
            pl.BlockSpec((T, d), lambda j: (0, 0)),
            pl.BlockSpec((vt, 1), lambda j: (j, 0)),
        ],
        out_specs=pl.BlockSpec((T, vt, B), lambda j: (0, j, 0)),
        out_shape=jax.ShapeDtypeStruct((T, V, B), jnp.float32),
        compiler_params=pltpu.CompilerParams(
            dimension_semantics=("parallel",),
            fuse_transposed_lhs_in_matmul=True,
        ),
    )(W, xT, pos, b2d)


def kernel(idx, tok_table, pos_table, W, b):
    B, T = idx.shape
    V, D = tok_table.shape
    n = B * T
    idx_t = idx.T.reshape(n)
    tok1 = tok_table.T.reshape(V * D)
    x_flat = _make_gather_t(n, T, D, V)(idx_t, tok1)
    xT = x_flat.reshape(T, D, B)
    out = _matmul_t(
        W, xT, pos_table[:T].astype(jnp.float32), b.reshape(V, 1)
    )
    return jnp.transpose(out, (2, 0, 1))


# speedup vs baseline: 3.0159x; 1.1687x over previous
import functools

import jax
import jax.numpy as jnp
from jax import lax
from jax.experimental import pallas as pl
from jax.experimental.pallas import tpu as pltpu
from jax.experimental.pallas import tpu_sc as plsc

_VOCAB_TILE = 3072
_L = 16
_CHUNK = 128


def _sc_info():
    try:
        info = plsc.get_sparse_core_info()
        return info.num_cores, info.num_subcores
    except Exception:
        return 2, 16


@functools.cache
def _make_gather_t(n, T, D, V):
    NC, NS = _sc_info()
    NW = NC * NS
    B = n // T
    slab_c = (D * T) // NW
    per_w = B * slab_c
    assert per_w % _CHUNK == 0 and B % _L == 0 and NW % T == 0
    w_per_t = NW // T
    mesh = plsc.VectorSubcoreMesh(core_axis_name="c", subcore_axis_name="s")

    @functools.partial(
        pl.kernel,
        mesh=mesh,
        out_type=jax.ShapeDtypeStruct((n * D,), jnp.float32),
        scratch_types=[
            pltpu.VMEM((B,), jnp.int32),
            pltpu.VMEM((per_w,), jnp.int32),
            pltpu.VMEM((per_w,), jnp.float32),
            pltpu.SemaphoreType.DMA,
        ],
        compiler_params=pltpu.CompilerParams(use_tc_tiling_on_sc=False),
    )
    def gather_t(idx_hbm, tok1_hbm, x_hbm, idx_v, ivec, x_s, sem):
        wid = lax.axis_index("s") * NC + lax.axis_index("c")
        t = wid // w_per_t
        c0 = (wid % w_per_t) * slab_c
        pltpu.sync_copy(idx_hbm.at[pl.ds(t * B, B)], idx_v)
        for cc in range(slab_c):
            cbase = (c0 + cc) * V
            for a in range(B // _L):
                i16 = idx_v[pl.ds(a * _L, _L)]
                ivec[pl.ds(cc * B + a * _L, _L)] = i16 + cbase
        copies = [
            pltpu.async_copy(
                tok1_hbm.at[ivec.at[pl.ds(k * _CHUNK, _CHUNK)]],
                x_s.at[pl.ds(k * _CHUNK, _CHUNK)],
                sem,
            )
            for k in range(per_w // _CHUNK)
        ]
        for cp in copies:
            cp.wait()
        pltpu.sync_copy(x_s, x_hbm.at[pl.ds(wid * per_w, per_w)])

    return gather_t


def _mm_body(w_ref, x_ref, pos_ref, b_ref, o_ref):
    wt = lax.transpose(w_ref[...], (1, 0))
    bias = b_ref[...].reshape(b_ref.shape[0], 1)
    for t in range(o_ref.shape[0]):
        acc = lax.dot_general(
            wt, x_ref[t],
            dimension_numbers=(((1,), (0,)), ((), ())),
            preferred_element_type=jnp.float32,
        )
        pw = lax.dot_general(
            wt, pos_ref[pl.ds(t, 1), :],
            dimension_numbers=(((1,), (1,)), ((), ())),
            preferred_element_type=jnp.float32,
        )
        o_ref[t] = acc + pw + bias


def _matmul_t(W, xT, pos, b1):
    d, V = W.shape
    T, _, B = xT.shape
    vt = _VOCAB_TILE
    return pl.pallas_call(
        _mm_body,
        grid=(pl.cdiv(V, vt),),
        in_specs=[
            pl.BlockSpec((d, vt), lambda j: (0, j)),
            pl.BlockSpec((T, d, B), lambda j: (0, 0, 0)),
            pl.BlockSpec((T, d), lambda j: (0, 0)),
            pl.BlockSpec((vt,), lambda j: (j,)),
        ],
        out_specs=pl.BlockSpec((T, vt, B), lambda j: (0, j, 0)),
        out_shape=jax.ShapeDtypeStruct((T, V, B), jnp.float32),
        compiler_params=pltpu.CompilerParams(
            dimension_semantics=("parallel",),
            fuse_transposed_lhs_in_matmul=True,
        ),
    )(W, xT, pos, b1)


def kernel(idx, tok_table, pos_table, W, b):
    B, T = idx.shape
    V, D = tok_table.shape
    n = B * T
    idx_t = idx.T.reshape(n)
    tok1 = tok_table.T.reshape(V * D)
    x_flat = _make_gather_t(n, T, D, V)(idx_t, tok1)
    xT = x_flat.reshape(T, D, B)
    out = _matmul_t(W, xT, pos_table[:T].astype(jnp.float32), b)
    return jnp.transpose(out, (2, 0, 1))
